# baseline (speedup 1.0000x reference)
import functools

import jax
import jax.numpy as jnp
from jax import lax
from jax.experimental import pallas as pl
from jax.experimental.pallas import tpu as pltpu
from jax.experimental.pallas import tpu_sc as plsc

NC = 2
NS = 16
NW = NC * NS
CHUNK = 128


def _zmat_body(x_ref, w_ref, z_ref):
    z_ref[...] = lax.dot_general(
        x_ref[...], w_ref[...], (((1,), (1,)), ((), ())),
        preferred_element_type=jnp.float32)


def _epilogue_body(a0_ref, a1_ref, b_ref, o_ref):
    t = a0_ref[0] + a1_ref[0]
    re0 = t[:, 0:16]
    re1 = t[:, 16:32]
    im1 = t[:, 32:48]
    m0 = jnp.sqrt(jnp.maximum(re0 * re0, 1e-12))
    m1 = jnp.sqrt(jnp.maximum(re1 * re1 + im1 * im1, 1e-12))
    logits = m0 + m1 + b_ref[...]
    mx = jnp.max(logits, axis=1, keepdims=True)
    lse = mx + jnp.log(jnp.sum(jnp.exp(logits - mx), axis=1, keepdims=True))
    o_ref[...] = logits - lse


def _make_sc_kernel(n_pad, ept, pc_rows):
    rows_per = n_pad // NS
    hpc = ept * CHUNK // 2
    mesh = plsc.VectorSubcoreMesh(core_axis_name="c", subcore_axis_name="s")

    @functools.partial(
        pl.kernel, mesh=mesh,
        compiler_params=pltpu.CompilerParams(use_tc_tiling_on_sc=False),
        out_type=jax.ShapeDtypeStruct((NC, n_pad, 48), jnp.float32),
        scratch_types=[
            pltpu.VMEM((ept + 2, CHUNK), jnp.int32),
            pltpu.VMEM((ept + 2, CHUNK), jnp.int32),
            pltpu.VMEM((hpc, 16), jnp.float32),
            pltpu.VMEM((CHUNK, 64), jnp.float32),
            pltpu.VMEM((CHUNK, 64), jnp.float32),
            pltpu.VMEM((CHUNK, 48), jnp.float32),
            pltpu.VMEM_SHARED((n_pad, 48), jnp.float32),
            pltpu.SemaphoreType.DMA,
            pltpu.SemaphoreType.DMA,
        ],
    )
    def sc_kernel(src_hbm, dst_hbm, pc_hbm, z_hbm, zeros_hbm, out_hbm,
                  siv, div, pva, zv0, zv1, mv, acc, gsem0, gsem1):
        c = lax.axis_index("c")
        s = lax.axis_index("s")
        wid = c * NS + s

        tile_row = pl.multiple_of(wid * ept, 8)
        pltpu.sync_copy(src_hbm.at[pl.ds(tile_row, ept + 2)], siv)
        pltpu.sync_copy(dst_hbm.at[pl.ds(tile_row, ept + 2)], div)
        pva_off = pl.multiple_of(
            jnp.minimum(wid * hpc, pc_rows - hpc), 8)
        local_pc = wid * hpc - pva_off
        pltpu.sync_copy(pc_hbm.at[pl.ds(pva_off, hpc)], pva)
        row0 = pl.multiple_of(s * rows_per, rows_per)
        pltpu.sync_copy(zeros_hbm.at[pl.ds(row0, rows_per)],
                        acc.at[pl.ds(row0, rows_per)])
        plsc.subcore_barrier()

        pltpu.async_copy(z_hbm.at[div.at[0]], zv0, gsem0)
        pltpu.async_copy(z_hbm.at[div.at[1]], zv1, gsem1)

        def pair_body(t, carry):
            for k, zvp, gsem in ((0, zv0, gsem0), (1, zv1, gsem1)):
                cr = 2 * t + k
                pltpu.make_async_copy(z_hbm.at[div.at[cr]], zvp, gsem).wait()
                pbase = jnp.minimum(local_pc + cr * (CHUNK // 2),
                                    hpc - CHUNK // 2)

                def edge_body(i, carry2):
                    row = pva[pbase + i, pl.ds(0, 16)]
                    for half in range(2):
                        e = 2 * i + half
                        q = 8 * half
                        z00 = zvp[e, pl.ds(0, 16)]
                        z01 = zvp[e, pl.ds(16, 16)]
                        z10 = zvp[e, pl.ds(32, 16)]
                        z11 = zvp[e, pl.ds(48, 16)]
                        mv[e, pl.ds(0, 16)] = (row[q + 0] * z00
                                               + row[q + 2] * z01)
                        mv[e, pl.ds(16, 16)] = (row[q + 4] * z10
                                                + row[q + 6] * z11)
                        mv[e, pl.ds(32, 16)] = (row[q + 5] * z10
                                                + row[q + 7] * z11)
                    return carry2

                lax.fori_loop(0, CHUNK // 2, edge_body, 0)
                pltpu.async_copy(z_hbm.at[div.at[cr + 2]], zvp, gsem)
                pltpu.sync_copy(mv, acc.at[siv.at[cr]], add=True)
            return carry

        lax.fori_loop(0, ept // 2, pair_body, 0)
        pltpu.make_async_copy(z_hbm.at[div.at[ept]], zv0, gsem0).wait()
        pltpu.make_async_copy(z_hbm.at[div.at[ept + 1]], zv1, gsem1).wait()
        plsc.subcore_barrier()

        pltpu.sync_copy(acc.at[pl.ds(row0, rows_per)],
                        out_hbm.at[c, pl.ds(row0, rows_per)])

    return sc_kernel


def kernel(x, edge_index, precomp, connection, radial_profile, bias):
    del connection
    n, f_in = x.shape
    e = edge_index.shape[1]

    ept = 2 * (-(-e // (NW * CHUNK * 2)))
    e_pad = ept * NW * CHUNK
    n_pad = -(-(n + 1) // (NS * 8)) * (NS * 8)
    pad = e_pad + 8 * CHUNK - e
    src = jnp.concatenate(
        [edge_index[0], jnp.full((pad,), n, jnp.int32)]).reshape(-1, CHUNK)
    dst = jnp.concatenate(
        [edge_index[1], jnp.zeros((pad,), jnp.int32)]).reshape(-1, CHUNK)
    pc = precomp.reshape(e // 2, 16)
    w = radial_profile[:2].reshape(64, f_in)
    zeros = jnp.zeros((n_pad, 48), jnp.float32)

    blk = 1000
    z = pl.pallas_call(
        _zmat_body,
        grid=(n // blk,),
        in_specs=[pl.BlockSpec((blk, f_in), lambda i: (i, 0)),
                  pl.BlockSpec((64, f_in), lambda i: (0, 0))],
        out_specs=pl.BlockSpec((blk, 64), lambda i: (i, 0)),
        out_shape=jax.ShapeDtypeStruct((n, 64), jnp.float32),
    )(x, w)

    acc2 = _make_sc_kernel(n_pad, ept, e // 2)(src, dst, pc, z, zeros)

    out = pl.pallas_call(
        _epilogue_body,
        grid=(n // blk,),
        in_specs=[pl.BlockSpec((1, blk, 48), lambda i: (0, i, 0)),
                  pl.BlockSpec((1, blk, 48), lambda i: (1, i, 0)),
                  pl.BlockSpec((1, 16), lambda i: (0, 0))],
        out_specs=pl.BlockSpec((blk, 16), lambda i: (i, 0)),
        out_shape=jax.ShapeDtypeStruct((n, 16), jnp.float32),
    )(acc2, acc2, bias.reshape(1, 16))
    return out


# speedup vs baseline: 1.0001x; 1.0001x over previous
import functools

import jax
import jax.numpy as jnp
from jax import lax
from jax.experimental import pallas as pl
from jax.experimental.pallas import tpu as pltpu
from jax.experimental.pallas import tpu_sc as plsc

NC = 2
NS = 16
NW = NC * NS
CHUNK = 128


def _zmat_body(x_ref, w_ref, z_ref):
    z_ref[...] = lax.dot_general(
        x_ref[...], w_ref[...], (((1,), (1,)), ((), ())),
        preferred_element_type=jnp.float32)


def _epilogue_body(a0_ref, a1_ref, b_ref, o_ref):
    t = a0_ref[0] + a1_ref[0]
    re0 = t[:, 0:16]
    re1 = t[:, 16:32]
    im1 = t[:, 32:48]
    m0 = jnp.sqrt(jnp.maximum(re0 * re0, 1e-12))
    m1 = jnp.sqrt(jnp.maximum(re1 * re1 + im1 * im1, 1e-12))
    logits = m0 + m1 + b_ref[...]
    mx = jnp.max(logits, axis=1, keepdims=True)
    lse = mx + jnp.log(jnp.sum(jnp.exp(logits - mx), axis=1, keepdims=True))
    o_ref[...] = logits - lse


def _make_sc_kernel(n_pad, ept, pc_rows):
    rows_per = n_pad // NS
    hpc = ept * CHUNK // 2
    mesh = plsc.VectorSubcoreMesh(core_axis_name="c", subcore_axis_name="s")

    @functools.partial(
        pl.kernel, mesh=mesh,
        compiler_params=pltpu.CompilerParams(use_tc_tiling_on_sc=False),
        out_type=jax.ShapeDtypeStruct((NC, n_pad, 48), jnp.float32),
        scratch_types=[
            pltpu.VMEM((ept + 2, CHUNK), jnp.int32),
            pltpu.VMEM((ept + 2, CHUNK), jnp.int32),
            pltpu.VMEM((hpc * 16,), jnp.float32),
            pltpu.VMEM((CHUNK, 64), jnp.float32),
            pltpu.VMEM((CHUNK, 64), jnp.float32),
            pltpu.VMEM((CHUNK, 48), jnp.float32),
            pltpu.VMEM_SHARED((n_pad, 48), jnp.float32),
            pltpu.SemaphoreType.DMA,
            pltpu.SemaphoreType.DMA,
        ],
    )
    def sc_kernel(src_hbm, dst_hbm, pc_hbm, z_hbm, zeros_hbm, out_hbm,
                  siv, div, pva, zv0, zv1, mv, acc, gsem0, gsem1):
        c = lax.axis_index("c")
        s = lax.axis_index("s")
        wid = c * NS + s

        tile_row = pl.multiple_of(wid * ept, 8)
        pltpu.sync_copy(src_hbm.at[pl.ds(tile_row, ept + 2)], siv)
        pltpu.sync_copy(dst_hbm.at[pl.ds(tile_row, ept + 2)], div)
        pva_off = jnp.minimum(wid * hpc, pc_rows - hpc)
        local_pc = wid * hpc - pva_off
        pltpu.sync_copy(
            pc_hbm.at[pl.ds(pl.multiple_of(pva_off * 16, 16), hpc * 16)], pva)
        row0 = pl.multiple_of(s * rows_per, rows_per)
        pltpu.sync_copy(zeros_hbm.at[pl.ds(row0, rows_per)],
                        acc.at[pl.ds(row0, rows_per)])
        plsc.subcore_barrier()

        pltpu.async_copy(z_hbm.at[div.at[0]], zv0, gsem0)
        pltpu.async_copy(z_hbm.at[div.at[1]], zv1, gsem1)

        def pair_body(t, carry):
            for k, zvp, gsem in ((0, zv0, gsem0), (1, zv1, gsem1)):
                cr = 2 * t + k
                pltpu.make_async_copy(z_hbm.at[div.at[cr]], zvp, gsem).wait()
                pbase = jnp.minimum(local_pc + cr * (CHUNK // 2),
                                    hpc - CHUNK // 2)

                def edge_body(i, carry2):
                    row = pva[pl.ds(pl.multiple_of((pbase + i) * 16, 16), 16)]
                    for half in range(2):
                        e = 2 * i + half
                        q = 8 * half
                        z00 = zvp[e, pl.ds(0, 16)]
                        z01 = zvp[e, pl.ds(16, 16)]
                        z10 = zvp[e, pl.ds(32, 16)]
                        z11 = zvp[e, pl.ds(48, 16)]
                        mv[e, pl.ds(0, 16)] = (row[q + 0] * z00
                                               + row[q + 2] * z01)
                        mv[e, pl.ds(16, 16)] = (row[q + 4] * z10
                                                + row[q + 6] * z11)
                        mv[e, pl.ds(32, 16)] = (row[q + 5] * z10
                                                + row[q + 7] * z11)
                    return carry2

                lax.fori_loop(0, CHUNK // 2, edge_body, 0)
                pltpu.async_copy(z_hbm.at[div.at[cr + 2]], zvp, gsem)
                pltpu.sync_copy(mv, acc.at[siv.at[cr]], add=True)
            return carry

        lax.fori_loop(0, ept // 2, pair_body, 0)
        pltpu.make_async_copy(z_hbm.at[div.at[ept]], zv0, gsem0).wait()
        pltpu.make_async_copy(z_hbm.at[div.at[ept + 1]], zv1, gsem1).wait()
        plsc.subcore_barrier()

        pltpu.sync_copy(acc.at[pl.ds(row0, rows_per)],
                        out_hbm.at[c, pl.ds(row0, rows_per)])

    return sc_kernel


def kernel(x, edge_index, precomp, connection, radial_profile, bias):
    del connection
    n, f_in = x.shape
    e = edge_index.shape[1]

    ept = 2 * (-(-e // (NW * CHUNK * 2)))
    e_pad = ept * NW * CHUNK
    n_pad = -(-(n + 1) // (NS * 8)) * (NS * 8)
    pad = e_pad + 8 * CHUNK - e
    src = jnp.concatenate(
        [edge_index[0], jnp.full((pad,), n, jnp.int32)]).reshape(-1, CHUNK)
    dst = jnp.concatenate(
        [edge_index[1], jnp.zeros((pad,), jnp.int32)]).reshape(-1, CHUNK)
    pc = jnp.ravel(precomp)
    w = radial_profile[:2].reshape(64, f_in)
    zeros = jnp.zeros((n_pad, 48), jnp.float32)

    blk = 1000
    z = pl.pallas_call(
        _zmat_body,
        grid=(n // blk,),
        in_specs=[pl.BlockSpec((blk, f_in), lambda i: (i, 0)),
                  pl.BlockSpec((64, f_in), lambda i: (0, 0))],
        out_specs=pl.BlockSpec((blk, 64), lambda i: (i, 0)),
        out_shape=jax.ShapeDtypeStruct((n, 64), jnp.float32),
    )(x, w)

    acc2 = _make_sc_kernel(n_pad, ept, e // 2)(src, dst, pc, z, zeros)

    out = pl.pallas_call(
        _epilogue_body,
        grid=(n // blk,),
        in_specs=[pl.BlockSpec((1, blk, 48), lambda i: (0, i, 0)),
                  pl.BlockSpec((1, blk, 48), lambda i: (1, i, 0)),
                  pl.BlockSpec((1, 16), lambda i: (0, 0))],
        out_specs=pl.BlockSpec((blk, 16), lambda i: (i, 0)),
        out_shape=jax.ShapeDtypeStruct((n, 16), jnp.float32),
    )(acc2, acc2, bias.reshape(1, 16))
    return out


# speedup vs baseline: 3.3768x; 3.3765x over previous
import functools

import jax
import jax.numpy as jnp
from jax import lax
from jax.experimental import pallas as pl
from jax.experimental.pallas import tpu as pltpu
from jax.experimental.pallas import tpu_sc as plsc

NC = 2
NS = 16
NW = NC * NS
CHUNK = 128


def _zmat_body(x_ref, w_ref, z_ref):
    z_ref[...] = lax.dot_general(
        x_ref[...], w_ref[...], (((1,), (1,)), ((), ())),
        preferred_element_type=jnp.float32)


def _epilogue_body(a0_ref, a1_ref, b_ref, o_ref):
    t = a0_ref[0] + a1_ref[0]
    re0 = t[:, 0:16]
    re1 = t[:, 16:32]
    im1 = t[:, 32:48]
    m0 = jnp.sqrt(jnp.maximum(re0 * re0, 1e-12))
    m1 = jnp.sqrt(jnp.maximum(re1 * re1 + im1 * im1, 1e-12))
    logits = m0 + m1 + b_ref[...]
    mx = jnp.max(logits, axis=1, keepdims=True)
    lse = mx + jnp.log(jnp.sum(jnp.exp(logits - mx), axis=1, keepdims=True))
    o_ref[...] = logits - lse


def _make_sc_kernel(n_pad, ept, pc_rows):
    rows_per = n_pad // NS
    hpc = ept * CHUNK // 2
    mesh = plsc.VectorSubcoreMesh(core_axis_name="c", subcore_axis_name="s")

    @functools.partial(
        pl.kernel, mesh=mesh,
        compiler_params=pltpu.CompilerParams(use_tc_tiling_on_sc=False),
        out_type=jax.ShapeDtypeStruct((NC, n_pad, 48), jnp.float32),
        scratch_types=[
            pltpu.VMEM((ept + 2, CHUNK), jnp.int32),
            pltpu.VMEM((ept + 2, CHUNK), jnp.int32),
            pltpu.VMEM((hpc * 16,), jnp.float32),
            pltpu.VMEM((CHUNK, 64), jnp.float32),
            pltpu.VMEM((CHUNK, 64), jnp.float32),
            pltpu.VMEM((CHUNK, 48), jnp.float32),
            pltpu.VMEM_SHARED((n_pad, 48), jnp.float32),
            pltpu.SemaphoreType.DMA,
            pltpu.SemaphoreType.DMA,
        ],
    )
    def sc_kernel(src_hbm, dst_hbm, pc_hbm, z_hbm, zeros_hbm, out_hbm,
                  siv, div, pva, zv0, zv1, mv, acc, gsem0, gsem1):
        c = lax.axis_index("c")
        s = lax.axis_index("s")
        wid = c * NS + s

        tile_row = pl.multiple_of(wid * ept, 8)
        pltpu.sync_copy(src_hbm.at[pl.ds(tile_row, ept + 2)], siv)
        pltpu.sync_copy(dst_hbm.at[pl.ds(tile_row, ept + 2)], div)
        pva_off = jnp.minimum(wid * hpc, pc_rows - hpc)
        local_pc = wid * hpc - pva_off
        pltpu.sync_copy(
            pc_hbm.at[pl.ds(pl.multiple_of(pva_off * 16, 16), hpc * 16)], pva)
        row0 = pl.multiple_of(s * rows_per, rows_per)
        pltpu.sync_copy(zeros_hbm.at[pl.ds(row0, rows_per)],
                        acc.at[pl.ds(row0, rows_per)])
        plsc.subcore_barrier()

        pltpu.async_copy(z_hbm.at[div.at[0]], zv0, gsem0)
        pltpu.async_copy(z_hbm.at[div.at[1]], zv1, gsem1)

        def pair_body(t, carry):
            for k, zvp, gsem in ((0, zv0, gsem0), (1, zv1, gsem1)):
                cr = 2 * t + k
                pltpu.make_async_copy(z_hbm.at[div.at[cr]], zvp, gsem).wait()
                pbase = jnp.minimum(local_pc + cr * (CHUNK // 2),
                                    hpc - CHUNK // 2)

                def edge_body(i, carry2):
                    row = pva[pl.ds(pl.multiple_of((pbase + i) * 16, 16), 16)]
                    for half in range(2):
                        e = 2 * i + half
                        q = 8 * half
                        z00 = zvp[e, pl.ds(0, 16)]
                        z01 = zvp[e, pl.ds(16, 16)]
                        z10 = zvp[e, pl.ds(32, 16)]
                        z11 = zvp[e, pl.ds(48, 16)]
                        mv[e, pl.ds(0, 16)] = (row[q + 0] * z00
                                               + row[q + 2] * z01)
                        mv[e, pl.ds(16, 16)] = (row[q + 4] * z10
                                                + row[q + 6] * z11)
                        mv[e, pl.ds(32, 16)] = (row[q + 5] * z10
                                                + row[q + 7] * z11)
                    return carry2

                lax.fori_loop(0, CHUNK // 2, edge_body, 0)
                pltpu.async_copy(z_hbm.at[div.at[cr + 2]], zvp, gsem)
                pltpu.sync_copy(mv, acc.at[siv.at[cr]], add=True)
            return carry

        lax.fori_loop(0, ept // 2, pair_body, 0)
        pltpu.make_async_copy(z_hbm.at[div.at[ept]], zv0, gsem0).wait()
        pltpu.make_async_copy(z_hbm.at[div.at[ept + 1]], zv1, gsem1).wait()
        plsc.subcore_barrier()

        pltpu.sync_copy(acc.at[pl.ds(row0, rows_per)],
                        out_hbm.at[c, pl.ds(row0, rows_per)])

    return sc_kernel


def kernel(x, edge_index, precomp, connection, radial_profile, bias):
    del connection
    n, f_in = x.shape
    e = edge_index.shape[1]

    ept = 2 * (-(-e // (NW * CHUNK * 2)))
    e_pad = ept * NW * CHUNK
    n_pad = -(-(n + 1) // (NS * 8)) * (NS * 8)
    pad = e_pad + 8 * CHUNK - e
    src = jnp.concatenate(
        [edge_index[0], jnp.full((pad,), n, jnp.int32)]).reshape(-1, CHUNK)
    dst = jnp.concatenate(
        [edge_index[1], jnp.zeros((pad,), jnp.int32)]).reshape(-1, CHUNK)
    pc = jnp.concatenate(
        [precomp.reshape(e, 8), jnp.zeros((pad, 8), jnp.float32)]).ravel()
    w = radial_profile[:2].reshape(64, f_in)
    zeros = jnp.zeros((n_pad, 48), jnp.float32)

    blk = 1000
    z = pl.pallas_call(
        _zmat_body,
        grid=(n // blk,),
        in_specs=[pl.BlockSpec((blk, f_in), lambda i: (i, 0)),
                  pl.BlockSpec((64, f_in), lambda i: (0, 0))],
        out_specs=pl.BlockSpec((blk, 64), lambda i: (i, 0)),
        out_shape=jax.ShapeDtypeStruct((n, 64), jnp.float32),
    )(x, w)

    acc2 = _make_sc_kernel(n_pad, ept, (e + pad) // 2)(src, dst, pc, z, zeros)

    out = pl.pallas_call(
        _epilogue_body,
        grid=(n // blk,),
        in_specs=[pl.BlockSpec((1, blk, 48), lambda i: (0, i, 0)),
                  pl.BlockSpec((1, blk, 48), lambda i: (1, i, 0)),
                  pl.BlockSpec((1, 16), lambda i: (0, 0))],
        out_specs=pl.BlockSpec((blk, 16), lambda i: (i, 0)),
        out_shape=jax.ShapeDtypeStruct((n, 16), jnp.float32),
    )(acc2, acc2, bias.reshape(1, 16))
    return out


# speedup vs baseline: 3.5255x; 1.0440x over previous
import functools

import jax
import jax.numpy as jnp
from jax import lax
from jax.experimental import pallas as pl
from jax.experimental.pallas import tpu as pltpu
from jax.experimental.pallas import tpu_sc as plsc

NC = 2
NS = 16
NW = NC * NS
CHUNK = 128


def _zmat_body(x_ref, w_ref, z_ref):
    z_ref[...] = lax.dot_general(
        x_ref[...], w_ref[...], (((1,), (1,)), ((), ())),
        preferred_element_type=jnp.float32)


def _epilogue_body(a0_ref, a1_ref, b_ref, o_ref):
    t = a0_ref[0] + a1_ref[0]
    re0 = t[:, 0:16]
    re1 = t[:, 16:32]
    im1 = t[:, 32:48]
    m0 = jnp.sqrt(jnp.maximum(re0 * re0, 1e-12))
    m1 = jnp.sqrt(jnp.maximum(re1 * re1 + im1 * im1, 1e-12))
    logits = m0 + m1 + b_ref[...]
    mx = jnp.max(logits, axis=1, keepdims=True)
    lse = mx + jnp.log(jnp.sum(jnp.exp(logits - mx), axis=1, keepdims=True))
    o_ref[...] = logits - lse


def _make_sc_kernel(n_pad, cka, ckb, pc_rows):
    rows_per = n_pad // NS
    hpc = cka * CHUNK // 2
    mesh = plsc.VectorSubcoreMesh(core_axis_name="c", subcore_axis_name="s")

    @functools.partial(
        pl.kernel, mesh=mesh,
        compiler_params=pltpu.CompilerParams(use_tc_tiling_on_sc=False),
        out_type=jax.ShapeDtypeStruct((NC, n_pad, 48), jnp.float32),
        scratch_types=[
            pltpu.VMEM((cka + 2, CHUNK), jnp.int32),
            pltpu.VMEM((cka + 2, CHUNK), jnp.int32),
            pltpu.VMEM((hpc * 16,), jnp.float32),
            pltpu.VMEM((CHUNK, 64), jnp.float32),
            pltpu.VMEM((CHUNK, 64), jnp.float32),
            pltpu.VMEM((CHUNK, 48), jnp.float32),
            pltpu.VMEM_SHARED((n_pad, 48), jnp.float32),
            pltpu.SemaphoreType.DMA,
            pltpu.SemaphoreType.DMA,
        ],
    )
    def sc_kernel(src_hbm, dst_hbm, pc_hbm, z_hbm, zeros_hbm, out_hbm,
                  siv, div, pva, zv0, zv1, mv, acc, gsem0, gsem1):
        c = lax.axis_index("c")
        s = lax.axis_index("s")
        nch = jnp.where(c == 0, cka, ckb)

        tile_row = pl.multiple_of(
            jnp.where(c == 0, s * cka, NS * cka + s * ckb), 8)
        pltpu.sync_copy(src_hbm.at[pl.ds(tile_row, cka + 2)], siv)
        pltpu.sync_copy(dst_hbm.at[pl.ds(tile_row, cka + 2)], div)
        pva_off = jnp.minimum(tile_row * (CHUNK // 2), pc_rows - hpc)
        local_pc = tile_row * (CHUNK // 2) - pva_off
        pltpu.sync_copy(
            pc_hbm.at[pl.ds(pl.multiple_of(pva_off * 16, 16), hpc * 16)], pva)
        row0 = pl.multiple_of(s * rows_per, rows_per)
        pltpu.sync_copy(zeros_hbm.at[pl.ds(row0, rows_per)],
                        acc.at[pl.ds(row0, rows_per)])
        plsc.subcore_barrier()

        pltpu.async_copy(z_hbm.at[div.at[0]], zv0, gsem0)
        pltpu.async_copy(z_hbm.at[div.at[1]], zv1, gsem1)

        def pair_body(t, carry):
            for k, zvp, gsem in ((0, zv0, gsem0), (1, zv1, gsem1)):
                cr = 2 * t + k
                pltpu.make_async_copy(z_hbm.at[div.at[cr]], zvp, gsem).wait()
                pbase = jnp.minimum(local_pc + cr * (CHUNK // 2),
                                    hpc - CHUNK // 2)

                def edge_body(i, carry2):
                    row = pva[pl.ds(pl.multiple_of((pbase + i) * 16, 16), 16)]
                    for half in range(2):
                        e = 2 * i + half
                        q = 8 * half
                        z00 = zvp[e, pl.ds(0, 16)]
                        z01 = zvp[e, pl.ds(16, 16)]
                        z10 = zvp[e, pl.ds(32, 16)]
                        z11 = zvp[e, pl.ds(48, 16)]
                        mv[e, pl.ds(0, 16)] = (row[q + 0] * z00
                                               + row[q + 2] * z01)
                        mv[e, pl.ds(16, 16)] = (row[q + 4] * z10
                                                + row[q + 6] * z11)
                        mv[e, pl.ds(32, 16)] = (row[q + 5] * z10
                                                + row[q + 7] * z11)
                    return carry2

                lax.fori_loop(0, CHUNK // 2, edge_body, 0)
                pltpu.async_copy(z_hbm.at[div.at[cr + 2]], zvp, gsem)
                pltpu.sync_copy(mv, acc.at[siv.at[cr]], add=True)
            return carry

        lax.fori_loop(0, nch // 2, pair_body, 0)
        pltpu.make_async_copy(z_hbm.at[div.at[nch]], zv0, gsem0).wait()
        pltpu.make_async_copy(z_hbm.at[div.at[nch + 1]], zv1, gsem1).wait()
        plsc.subcore_barrier()

        pltpu.sync_copy(acc.at[pl.ds(row0, rows_per)],
                        out_hbm.at[c, pl.ds(row0, rows_per)])

    return sc_kernel


def kernel(x, edge_index, precomp, connection, radial_profile, bias):
    del connection
    n, f_in = x.shape
    e = edge_index.shape[1]

    ept = 2 * (-(-e // (NW * CHUNK * 2)))
    e_pad = ept * NW * CHUNK
    n_pad = -(-(n + 1) // (NS * 8)) * (NS * 8)
    per_s = 2 * ept
    cka = max(8, min((per_s * 3 // 5) // 8 * 8, per_s - 8))
    ckb = per_s - cka
    pad = e_pad + 24 * CHUNK - e
    src = jnp.concatenate(
        [edge_index[0], jnp.full((pad,), n, jnp.int32)]).reshape(-1, CHUNK)
    dst = jnp.concatenate(
        [edge_index[1], jnp.zeros((pad,), jnp.int32)]).reshape(-1, CHUNK)
    pc = jnp.concatenate(
        [precomp.reshape(e, 8), jnp.zeros((pad, 8), jnp.float32)]).ravel()
    w = radial_profile[:2].reshape(64, f_in)
    zeros = jnp.zeros((n_pad, 48), jnp.float32)

    blk = 1000
    z = pl.pallas_call(
        _zmat_body,
        grid=(n // blk,),
        in_specs=[pl.BlockSpec((blk, f_in), lambda i: (i, 0)),
                  pl.BlockSpec((64, f_in), lambda i: (0, 0))],
        out_specs=pl.BlockSpec((blk, 64), lambda i: (i, 0)),
        out_shape=jax.ShapeDtypeStruct((n, 64), jnp.float32),
    )(x, w)

    acc2 = _make_sc_kernel(n_pad, cka, ckb,
                           (e + pad) // 2)(src, dst, pc, z, zeros)

    out = pl.pallas_call(
        _epilogue_body,
        grid=(n // blk,),
        in_specs=[pl.BlockSpec((1, blk, 48), lambda i: (0, i, 0)),
                  pl.BlockSpec((1, blk, 48), lambda i: (1, i, 0)),
                  pl.BlockSpec((1, 16), lambda i: (0, 0))],
        out_specs=pl.BlockSpec((blk, 16), lambda i: (i, 0)),
        out_shape=jax.ShapeDtypeStruct((n, 16), jnp.float32),
    )(acc2, acc2, bias.reshape(1, 16))
    return out
